# baseline (speedup 1.0000x reference)
import jax
import jax.numpy as jnp
import numpy as np
from jax.experimental import pallas as pl
from jax.experimental.pallas import tpu as pltpu

_B, _S, _D = 1, 8192, 768
_E, _R, _O = 64, 16, 768
_ER = _E * _R
_SCALING = 32.0 / 16.0
_BT = 512


def _fused_body(x_ref, rw_ref, w_ref, bias_ref, af_ref, bf_ref, out_ref):
    xb = x_ref[...]
    z = jax.lax.dot_general(
        xb, af_ref[...], (((1,), (1,)), ((), ())),
        preferred_element_type=jnp.float32)
    base = jax.lax.dot_general(
        xb, w_ref[...], (((1,), (1,)), ((), ())),
        preferred_element_type=jnp.float32)
    logits = jax.lax.dot_general(
        xb, rw_ref[...], (((1,), (1,)), ((), ())),
        preferred_element_type=jnp.float32)
    iota_e = jax.lax.broadcasted_iota(jnp.int32, (_BT, _E), 1)
    m1 = jnp.max(logits, axis=1, keepdims=True)
    a1 = jnp.min(jnp.where(logits == m1, iota_e, _E), axis=1, keepdims=True)
    lm = jnp.where(iota_e == a1, -jnp.inf, logits)
    m2 = jnp.max(lm, axis=1, keepdims=True)
    a2 = jnp.min(jnp.where(lm == m2, iota_e, _E), axis=1, keepdims=True)
    e21 = jnp.exp(m2 - m1)
    p1 = 1.0 / (1.0 + e21)
    p2 = e21 / (1.0 + e21)
    col_e = jax.lax.broadcasted_iota(jnp.int32, (_BT, _ER), 1) // _R
    comb = jnp.where(col_e == a1, p1, 0.0) + jnp.where(col_e == a2, p2, 0.0)
    zm = z * comb
    y = jax.lax.dot_general(
        zm, bf_ref[...], (((1,), (0,)), ((), ())),
        preferred_element_type=jnp.float32)
    out_ref[...] = base + bias_ref[...] + _SCALING * y


def kernel(x, base_weight, base_bias, router_weight, lora_A, lora_B):
    orig_shape = x.shape
    xf = x.reshape(-1, x.shape[-1])
    n = xf.shape[0]
    a_flat = lora_A.reshape(_ER, _D)
    b_flat = lora_B.transpose(0, 2, 1).reshape(_ER, _O)
    bias2 = base_bias.reshape(1, _O)
    grid = (n // _BT,)
    out = pl.pallas_call(
        _fused_body,
        grid=grid,
        in_specs=[
            pl.BlockSpec((_BT, _D), lambda i: (i, 0)),
            pl.BlockSpec((_E, _D), lambda i: (0, 0)),
            pl.BlockSpec((_O, _D), lambda i: (0, 0)),
            pl.BlockSpec((1, _O), lambda i: (0, 0)),
            pl.BlockSpec((_ER, _D), lambda i: (0, 0)),
            pl.BlockSpec((_ER, _O), lambda i: (0, 0)),
        ],
        out_specs=pl.BlockSpec((_BT, _O), lambda i: (i, 0)),
        out_shape=jax.ShapeDtypeStruct((n, _O), jnp.float32),
        compiler_params=pltpu.CompilerParams(
            dimension_semantics=("parallel",)),
    )(xf, router_weight, base_weight, bias2, a_flat, b_flat)
    return out.reshape(*orig_shape[:-1], _O)


# speedup vs baseline: 1.3060x; 1.3060x over previous
import jax
import jax.numpy as jnp
from jax.experimental import pallas as pl
from jax.experimental.pallas import tpu as pltpu

_B, _S, _D = 1, 8192, 768
_E, _R, _O = 64, 16, 768
_ER = _E * _R
_SCALING = 32.0 / 16.0
_BT = 1024


def _fused_body(x_ref, rw_ref, w_ref, bias_ref, af_ref, bf_ref, out_ref):
    xb = x_ref[...]
    logits = jax.lax.dot_general(
        xb, rw_ref[...], (((1,), (1,)), ((), ())),
        preferred_element_type=jnp.float32)
    iota_e = jax.lax.broadcasted_iota(jnp.int32, (_BT, _E), 1)
    m1 = jnp.max(logits, axis=1, keepdims=True)
    a1 = jnp.min(jnp.where(logits == m1, iota_e, _E), axis=1, keepdims=True)
    lm = jnp.where(iota_e == a1, -jnp.inf, logits)
    m2 = jnp.max(lm, axis=1, keepdims=True)
    a2 = jnp.min(jnp.where(lm == m2, iota_e, _E), axis=1, keepdims=True)
    e21 = jnp.exp(m2 - m1)
    p1 = 1.0 / (1.0 + e21)
    p2 = e21 / (1.0 + e21)
    col_e = jax.lax.broadcasted_iota(jnp.int32, (_BT, _ER), 1) // _R
    comb = jnp.where(col_e == a1, p1, 0.0) + jnp.where(col_e == a2, p2, 0.0)
    z = jax.lax.dot_general(
        xb, af_ref[...], (((1,), (1,)), ((), ())),
        preferred_element_type=jnp.float32)
    zm = z * comb
    y = jax.lax.dot_general(
        zm, bf_ref[...], (((1,), (0,)), ((), ())),
        preferred_element_type=jnp.float32)
    base = jax.lax.dot_general(
        xb, w_ref[...], (((1,), (1,)), ((), ())),
        preferred_element_type=jnp.float32)
    out_ref[...] = base + bias_ref[...] + _SCALING * y


def kernel(x, base_weight, base_bias, router_weight, lora_A, lora_B):
    orig_shape = x.shape
    xf = x.reshape(-1, x.shape[-1])
    n = xf.shape[0]
    a_flat = lora_A.reshape(_ER, _D)
    b_flat = lora_B.transpose(0, 2, 1).reshape(_ER, _O)
    bias2 = base_bias.reshape(1, _O)
    grid = (n // _BT,)
    out = pl.pallas_call(
        _fused_body,
        grid=grid,
        in_specs=[
            pl.BlockSpec((_BT, _D), lambda i: (i, 0)),
            pl.BlockSpec((_E, _D), lambda i: (0, 0)),
            pl.BlockSpec((_O, _D), lambda i: (0, 0)),
            pl.BlockSpec((1, _O), lambda i: (0, 0)),
            pl.BlockSpec((_ER, _D), lambda i: (0, 0)),
            pl.BlockSpec((_ER, _O), lambda i: (0, 0)),
        ],
        out_specs=pl.BlockSpec((_BT, _O), lambda i: (i, 0)),
        out_shape=jax.ShapeDtypeStruct((n, _O), jnp.float32),
        compiler_params=pltpu.CompilerParams(
            dimension_semantics=("parallel",)),
    )(xf, router_weight, base_weight, bias2, a_flat, b_flat)
    return out.reshape(*orig_shape[:-1], _O)
